# baseline (device time: 141670 ns/iter reference)
import functools

import numpy as np
import jax
import jax.numpy as jnp
from jax import lax
from jax.experimental import pallas as pl
from jax.experimental.pallas import tpu as pltpu

N_DEV = 4
B, SQ, D = 2, 512, 1024
R = B * SQ
HL = 8
DH = 128
SCALE = 0.08838834764831843


def _rope_tables():
    inv = 1.0 / (10000.0 ** (np.arange(0, DH, 2) / DH))
    pos = np.arange(SQ)[:, None] * inv[None, :]
    cos = np.repeat(np.cos(pos), 2, axis=-1)
    sin = np.repeat(np.sin(pos), 2, axis=-1)
    even = (np.arange(DH) % 2 == 0).astype(np.float64)
    sin_a = -sin * even
    sin_b = sin * (1.0 - even)
    tile = lambda t: np.tile(t, (B, HL))
    return tile(cos), tile(sin_a), tile(sin_b)


def _body(x_ref, wq_ref, wk_ref, wv_ref, wo_ref, cos_ref, sa_ref, sb_ref,
          out_ref, xg, pout, rsbuf, qb, kb, vb, ctxb,
          ag_send, ag_recv, rs_send, rs_recv):
    my = lax.axis_index("i")
    right = lax.rem(my + 1, N_DEV)
    left = lax.rem(my + N_DEV - 1, N_DEV)

    bar = pltpu.get_barrier_semaphore()
    for nbr in (left, right):
        pl.semaphore_signal(bar, inc=1, device_id=(nbr,),
                            device_id_type=pl.DeviceIdType.MESH)
    pl.semaphore_wait(bar, 2)


    def ag_hop(h, src, dev):
        return pltpu.make_async_remote_copy(
            src_ref=src,
            dst_ref=xg.at[h],
            send_sem=ag_send.at[h],
            recv_sem=ag_recv.at[h],
            device_id=(dev,),
            device_id_type=pl.DeviceIdType.MESH,
        )

    def rs_hop(s, src_slot):
        return pltpu.make_async_remote_copy(
            src_ref=pout.at[src_slot],
            dst_ref=rsbuf.at[s],
            send_sem=rs_send.at[s],
            recv_sem=rs_recv.at[s],
            device_id=(left,),
            device_id_type=pl.DeviceIdType.MESH,
        )

    def rope(t):
        tl = jnp.concatenate([t[:, 1:], t[:, :1]], axis=1)
        tr = jnp.concatenate([t[:, -1:], t[:, :-1]], axis=1)
        return t * cos_ref[...] + tl * sa_ref[...] + tr * sb_ref[...]

    def compute_chunk(xc, c):
        q = jnp.dot(xc, wq_ref[...],
                    preferred_element_type=jnp.float32).astype(jnp.bfloat16)
        qb[...] = rope(q)
        k = jnp.dot(xc, wk_ref[...],
                    preferred_element_type=jnp.float32).astype(jnp.bfloat16)
        kb[...] = rope(k)
        vb[...] = jnp.dot(
            xc, wv_ref[...],
            preferred_element_type=jnp.float32).astype(jnp.bfloat16)

        def attn_b(b, _):
            rs = pl.ds(b * SQ, SQ)
            for h in range(HL):
                cs = pl.ds(h * DH, DH)
                qh = qb[rs, cs]
                kh = kb[rs, cs]
                s = lax.dot_general(
                    qh, kh, (((1,), (1,)), ((), ())),
                    preferred_element_type=jnp.float32,
                )
                w = jnp.exp(s)
                r = 1.0 / jnp.sum(w, axis=1, keepdims=True)
                ctx = jnp.dot(w.astype(jnp.bfloat16), vb[rs, cs],
                              preferred_element_type=jnp.float32)
                ctxb[rs, cs] = (ctx * r).astype(jnp.bfloat16)
            return 0

        lax.fori_loop(0, B, attn_b, 0)
        pout[c] = jnp.dot(
            ctxb[...], wo_ref[...], preferred_element_type=jnp.float32
        ).astype(jnp.bfloat16)

    ag_a = ag_hop(0, x_ref, right)
    ag_b = ag_hop(1, x_ref, left)
    ag_a.start()
    ag_b.start()
    compute_chunk(x_ref[...], 3)

    ag_a.wait_recv()
    ag_c = ag_hop(2, xg.at[0], right)
    ag_c.start()
    ag_b.wait_recv()
    compute_chunk(xg[1], 1)
    rs0 = rs_hop(0, 1)
    rs0.start()

    ag_c.wait_recv()
    compute_chunk(xg[2], 2)
    rs0.wait_recv()
    pout[2] = pout[2] + rsbuf[0]
    rs1 = rs_hop(1, 2)
    rs1.start()

    compute_chunk(xg[0], 0)
    rs1.wait_recv()
    pout[0] = pout[0] + rsbuf[1]
    rs2 = rs_hop(2, 0)
    rs2.start()

    rs2.wait_recv()
    out_ref[...] = (pout[3] + rsbuf[2]).reshape(B, SQ, D)

    for d in (ag_a, ag_b, ag_c, rs0, rs1, rs2):
        d.wait_send()

    @functools.partial(pl.run_scoped, sem=pltpu.SemaphoreType.REGULAR)
    def _(sem):
        for nbr in (left, right):
            pl.semaphore_signal(sem, inc=1, device_id=(nbr,),
                                device_id_type=pl.DeviceIdType.MESH)
        pl.semaphore_wait(sem, 2)


def kernel(x, Wq, Wk, Wv, Wo):
    cosf, sin_a, sin_b = _rope_tables()
    bf = jnp.bfloat16
    args = (
        x.astype(bf).reshape(R, D),
        (Wq * SCALE).astype(bf),
        Wk.astype(bf),
        Wv.astype(bf),
        Wo.astype(bf),
        jnp.asarray(cosf, dtype=bf),
        jnp.asarray(sin_a, dtype=bf),
        jnp.asarray(sin_b, dtype=bf),
    )
    out = pl.pallas_call(
        _body,
        out_shape=jax.ShapeDtypeStruct((B, SQ, D), bf),
        in_specs=[pl.BlockSpec(memory_space=pltpu.VMEM)] * len(args),
        out_specs=pl.BlockSpec(memory_space=pltpu.VMEM),
        scratch_shapes=[
            pltpu.VMEM((N_DEV - 1, R, D), jnp.bfloat16),
            pltpu.VMEM((N_DEV, R, D), jnp.bfloat16),
            pltpu.VMEM((N_DEV - 1, R, D), jnp.bfloat16),
            pltpu.VMEM((R, D), jnp.bfloat16),
            pltpu.VMEM((R, D), jnp.bfloat16),
            pltpu.VMEM((R, D), jnp.bfloat16),
            pltpu.VMEM((R, D), jnp.bfloat16),
            pltpu.SemaphoreType.DMA((N_DEV - 1,)),
            pltpu.SemaphoreType.DMA((N_DEV - 1,)),
            pltpu.SemaphoreType.DMA((N_DEV - 1,)),
            pltpu.SemaphoreType.DMA((N_DEV - 1,)),
        ],
        compiler_params=pltpu.CompilerParams(
            collective_id=0,
            vmem_limit_bytes=100 * 1024 * 1024,
        ),
    )(*args)
    return out


if __name__ == "__main__":
    t = np.random.randn(R, D).astype(np.float64)
    cos, sa, sb = _rope_tables()
    tl = np.concatenate([t[:, 1:], t[:, :1]], axis=1)
    tr = np.concatenate([t[:, -1:], t[:, :-1]], axis=1)
    mine = t * cos + tl * sa + tr * sb

    t4 = t.reshape(B, SQ, HL, DH)
    inv = 1.0 / (10000.0 ** (np.arange(0, DH, 2) / DH))
    pos = np.arange(SQ)[:, None] * inv[None, :]
    cos_r = np.repeat(np.cos(pos), 2, axis=-1)
    sin_r = np.repeat(np.sin(pos), 2, axis=-1)
    t2 = t4.reshape(B, SQ, HL, DH // 2, 2)
    t_r = np.stack([-t2[..., 1], t2[..., 0]], axis=-1).reshape(B, SQ, HL, DH)
    ref = t4 * cos_r[None, :, None, :] + t_r * sin_r[None, :, None, :]
    print("rope table max err:", np.abs(mine - ref.reshape(R, D)).max())


# device time: 141656 ns/iter; 1.0001x vs baseline; 1.0001x over previous
import functools

import numpy as np
import jax
import jax.numpy as jnp
from jax import lax
from jax.experimental import pallas as pl
from jax.experimental.pallas import tpu as pltpu

N_DEV = 4
B, SQ, D = 2, 512, 1024
R = B * SQ
HL = 8
DH = 128
SCALE = 0.08838834764831843


def _rope_tables():
    inv = 1.0 / (10000.0 ** (np.arange(0, DH, 2) / DH))
    pos = np.arange(SQ)[:, None] * inv[None, :]
    cos = np.repeat(np.cos(pos), 2, axis=-1)
    sin = np.repeat(np.sin(pos), 2, axis=-1)
    even = (np.arange(DH) % 2 == 0).astype(np.float64)
    sin_a = -sin * even
    sin_b = sin * (1.0 - even)
    tile = lambda t: np.tile(t, (B, HL))
    return tile(cos), tile(sin_a), tile(sin_b)


def _body(x_ref, wq_ref, wk_ref, wv_ref, wo_ref, cos_ref, sa_ref, sb_ref,
          out_ref, xg, pout, rsbuf, qb, kb, vb, ctxb,
          ag_send, ag_recv, rs_send, rs_recv):
    my = lax.axis_index("i")
    right = lax.rem(my + 1, N_DEV)
    left = lax.rem(my + N_DEV - 1, N_DEV)

    bar = pltpu.get_barrier_semaphore()
    for nbr in (left, right):
        pl.semaphore_signal(bar, inc=1, device_id=(nbr,),
                            device_id_type=pl.DeviceIdType.MESH)
    pl.semaphore_wait(bar, 2)


    def ag_hop(h, src, dev):
        return pltpu.make_async_remote_copy(
            src_ref=src,
            dst_ref=xg.at[h],
            send_sem=ag_send.at[h],
            recv_sem=ag_recv.at[h],
            device_id=(dev,),
            device_id_type=pl.DeviceIdType.MESH,
        )

    def rs_hop(s, src_slot):
        return pltpu.make_async_remote_copy(
            src_ref=pout.at[src_slot],
            dst_ref=rsbuf.at[s],
            send_sem=rs_send.at[s],
            recv_sem=rs_recv.at[s],
            device_id=(left,),
            device_id_type=pl.DeviceIdType.MESH,
        )

    def rope(t):
        tl = jnp.concatenate([t[:, 1:], t[:, :1]], axis=1)
        tr = jnp.concatenate([t[:, -1:], t[:, :-1]], axis=1)
        return t * cos_ref[...] + tl * sa_ref[...] + tr * sb_ref[...]

    def compute_chunk(xc, c):
        q = jnp.dot(xc, wq_ref[...],
                    preferred_element_type=jnp.float32).astype(jnp.bfloat16)
        qb[...] = rope(q)
        k = jnp.dot(xc, wk_ref[...],
                    preferred_element_type=jnp.float32).astype(jnp.bfloat16)
        kb[...] = rope(k)
        vb[...] = jnp.dot(
            xc, wv_ref[...],
            preferred_element_type=jnp.float32).astype(jnp.bfloat16)

        def attn_b(b, _):
            rs = pl.ds(b * SQ, SQ)
            for h in range(HL):
                cs = pl.ds(h * DH, DH)
                qh = qb[rs, cs]
                kh = kb[rs, cs]
                s = lax.dot_general(
                    qh, kh, (((1,), (1,)), ((), ())),
                    preferred_element_type=jnp.float32,
                )
                w = s
                r = 1.0 / jnp.sum(w, axis=1, keepdims=True)
                ctx = jnp.dot(w.astype(jnp.bfloat16), vb[rs, cs],
                              preferred_element_type=jnp.float32)
                ctxb[rs, cs] = (ctx * r).astype(jnp.bfloat16)
            return 0

        lax.fori_loop(0, B, attn_b, 0)
        pout[c] = jnp.dot(
            ctxb[...], wo_ref[...], preferred_element_type=jnp.float32
        ).astype(jnp.bfloat16)

    ag_a = ag_hop(0, x_ref, right)
    ag_b = ag_hop(1, x_ref, left)
    ag_a.start()
    ag_b.start()
    compute_chunk(x_ref[...], 3)

    ag_a.wait_recv()
    ag_c = ag_hop(2, xg.at[0], right)
    ag_c.start()
    ag_b.wait_recv()
    compute_chunk(xg[1], 1)
    rs0 = rs_hop(0, 1)
    rs0.start()

    ag_c.wait_recv()
    compute_chunk(xg[2], 2)
    rs0.wait_recv()
    pout[2] = pout[2] + rsbuf[0]
    rs1 = rs_hop(1, 2)
    rs1.start()

    compute_chunk(xg[0], 0)
    rs1.wait_recv()
    pout[0] = pout[0] + rsbuf[1]
    rs2 = rs_hop(2, 0)
    rs2.start()

    rs2.wait_recv()
    out_ref[...] = (pout[3] + rsbuf[2]).reshape(B, SQ, D)

    for d in (ag_a, ag_b, ag_c, rs0, rs1, rs2):
        d.wait_send()

    @functools.partial(pl.run_scoped, sem=pltpu.SemaphoreType.REGULAR)
    def _(sem):
        for nbr in (left, right):
            pl.semaphore_signal(sem, inc=1, device_id=(nbr,),
                                device_id_type=pl.DeviceIdType.MESH)
        pl.semaphore_wait(sem, 2)


def kernel(x, Wq, Wk, Wv, Wo):
    cosf, sin_a, sin_b = _rope_tables()
    bf = jnp.bfloat16
    args = (
        x.astype(bf).reshape(R, D),
        (Wq * SCALE).astype(bf),
        Wk.astype(bf),
        Wv.astype(bf),
        Wo.astype(bf),
        jnp.asarray(cosf, dtype=bf),
        jnp.asarray(sin_a, dtype=bf),
        jnp.asarray(sin_b, dtype=bf),
    )
    out = pl.pallas_call(
        _body,
        out_shape=jax.ShapeDtypeStruct((B, SQ, D), bf),
        in_specs=[pl.BlockSpec(memory_space=pltpu.VMEM)] * len(args),
        out_specs=pl.BlockSpec(memory_space=pltpu.VMEM),
        scratch_shapes=[
            pltpu.VMEM((N_DEV - 1, R, D), jnp.bfloat16),
            pltpu.VMEM((N_DEV, R, D), jnp.bfloat16),
            pltpu.VMEM((N_DEV - 1, R, D), jnp.bfloat16),
            pltpu.VMEM((R, D), jnp.bfloat16),
            pltpu.VMEM((R, D), jnp.bfloat16),
            pltpu.VMEM((R, D), jnp.bfloat16),
            pltpu.VMEM((R, D), jnp.bfloat16),
            pltpu.SemaphoreType.DMA((N_DEV - 1,)),
            pltpu.SemaphoreType.DMA((N_DEV - 1,)),
            pltpu.SemaphoreType.DMA((N_DEV - 1,)),
            pltpu.SemaphoreType.DMA((N_DEV - 1,)),
        ],
        compiler_params=pltpu.CompilerParams(
            collective_id=0,
            vmem_limit_bytes=100 * 1024 * 1024,
        ),
    )(*args)
    return out


if __name__ == "__main__":
    t = np.random.randn(R, D).astype(np.float64)
    cos, sa, sb = _rope_tables()
    tl = np.concatenate([t[:, 1:], t[:, :1]], axis=1)
    tr = np.concatenate([t[:, -1:], t[:, :-1]], axis=1)
    mine = t * cos + tl * sa + tr * sb

    t4 = t.reshape(B, SQ, HL, DH)
    inv = 1.0 / (10000.0 ** (np.arange(0, DH, 2) / DH))
    pos = np.arange(SQ)[:, None] * inv[None, :]
    cos_r = np.repeat(np.cos(pos), 2, axis=-1)
    sin_r = np.repeat(np.sin(pos), 2, axis=-1)
    t2 = t4.reshape(B, SQ, HL, DH // 2, 2)
    t_r = np.stack([-t2[..., 1], t2[..., 0]], axis=-1).reshape(B, SQ, HL, DH)
    ref = t4 * cos_r[None, :, None, :] + t_r * sin_r[None, :, None, :]
    print("rope table max err:", np.abs(mine - ref.reshape(R, D)).max())


# device time: 138224 ns/iter; 1.0249x vs baseline; 1.0248x over previous
import functools

import numpy as np
import jax
import jax.numpy as jnp
from jax import lax
from jax.experimental import pallas as pl
from jax.experimental.pallas import tpu as pltpu

N_DEV = 4
B, SQ, D = 2, 512, 1024
R = B * SQ
HL = 8
DH = 128
SCALE = 0.08838834764831843


def _rope_tables():
    inv = 1.0 / (10000.0 ** (np.arange(0, DH, 2) / DH))
    pos = np.arange(SQ)[:, None] * inv[None, :]
    cos = np.repeat(np.cos(pos), 2, axis=-1)
    sin = np.repeat(np.sin(pos), 2, axis=-1)
    even = (np.arange(DH) % 2 == 0).astype(np.float64)
    sin_a = -sin * even
    sin_b = sin * (1.0 - even)
    tile = lambda t: np.tile(t, (B, HL))
    return tile(cos), tile(sin_a), tile(sin_b)


def _body(x_ref, wq_ref, wk_ref, wv_ref, wo_ref, cos_ref, sa_ref, sb_ref,
          out_ref, xg, pout, rsbuf, qb, kb, vb, ctxb,
          ag_send, ag_recv, rs_send, rs_recv):
    my = lax.axis_index("i")
    right = lax.rem(my + 1, N_DEV)
    left = lax.rem(my + N_DEV - 1, N_DEV)

    bar = pltpu.get_barrier_semaphore()
    for nbr in (left, right):
        pl.semaphore_signal(bar, inc=1, device_id=(nbr,),
                            device_id_type=pl.DeviceIdType.MESH)
    pl.semaphore_wait(bar, 2)


    def ag_hop(h, src, dev):
        return pltpu.make_async_remote_copy(
            src_ref=src,
            dst_ref=xg.at[h],
            send_sem=ag_send.at[h],
            recv_sem=ag_recv.at[h],
            device_id=(dev,),
            device_id_type=pl.DeviceIdType.MESH,
        )

    def rs_hop(s, j, c):
        k = 2 * s + j
        return pltpu.make_async_remote_copy(
            src_ref=pout.at[2 * c + j],
            dst_ref=rsbuf.at[k],
            send_sem=rs_send.at[k],
            recv_sem=rs_recv.at[k],
            device_id=(left,),
            device_id_type=pl.DeviceIdType.MESH,
        )

    def rope(t):
        tl = jnp.concatenate([t[:, 1:], t[:, :1]], axis=1)
        tr = jnp.concatenate([t[:, -1:], t[:, :-1]], axis=1)
        return t * cos_ref[...] + tl * sa_ref[...] + tr * sb_ref[...]

    def qkv_attn(xc):
        q = jnp.dot(xc, wq_ref[...],
                    preferred_element_type=jnp.float32).astype(jnp.bfloat16)
        qb[...] = rope(q)
        k = jnp.dot(xc, wk_ref[...],
                    preferred_element_type=jnp.float32).astype(jnp.bfloat16)
        kb[...] = rope(k)
        vb[...] = jnp.dot(
            xc, wv_ref[...],
            preferred_element_type=jnp.float32).astype(jnp.bfloat16)

        def attn_b(b, _):
            rs = pl.ds(b * SQ, SQ)
            for h in range(HL):
                cs = pl.ds(h * DH, DH)
                qh = qb[rs, cs]
                kh = kb[rs, cs]
                s = lax.dot_general(
                    qh, kh, (((1,), (1,)), ((), ())),
                    preferred_element_type=jnp.float32,
                )
                w = jnp.exp(s)
                r = 1.0 / jnp.sum(w, axis=1, keepdims=True)
                ctx = jnp.dot(w.astype(jnp.bfloat16), vb[rs, cs],
                              preferred_element_type=jnp.float32)
                ctxb[rs, cs] = (ctx * r).astype(jnp.bfloat16)
            return 0

        lax.fori_loop(0, B, attn_b, 0)

    def oproj_half(j):
        rows = pl.ds(j * SQ, SQ)
        return jnp.dot(ctxb[rows, :], wo_ref[...],
                       preferred_element_type=jnp.float32).astype(jnp.bfloat16)

    def rs_chunk(c, s, prev):
        descs = []
        for j in (0, 1):
            ph = oproj_half(j)
            if prev is not None:
                prev[j].wait_recv()
                ph = ph + rsbuf[2 * (s - 1) + j]
            pout[2 * c + j] = ph
            d = rs_hop(s, j, c)
            d.start()
            descs.append(d)
        return descs

    ag_a = ag_hop(0, x_ref, right)
    ag_b = ag_hop(1, x_ref, left)
    ag_a.start()
    ag_b.start()
    qkv_attn(x_ref[...])
    pout[6] = oproj_half(0)
    pout[7] = oproj_half(1)

    ag_a.wait_recv()
    ag_c = ag_hop(2, xg.at[0], right)
    ag_c.start()
    ag_b.wait_recv()
    qkv_attn(xg[1])
    rs0 = rs_chunk(1, 0, None)

    ag_c.wait_recv()
    qkv_attn(xg[2])
    rs1 = rs_chunk(2, 1, rs0)

    qkv_attn(xg[0])
    rs2 = rs_chunk(0, 2, rs1)

    rs2[0].wait_recv()
    out_ref[0] = pout[6] + rsbuf[4]
    rs2[1].wait_recv()
    out_ref[1] = pout[7] + rsbuf[5]

    for d in (ag_a, ag_b, ag_c, *rs0, *rs1, *rs2):
        d.wait_send()

    @functools.partial(pl.run_scoped, sem=pltpu.SemaphoreType.REGULAR)
    def _(sem):
        for nbr in (left, right):
            pl.semaphore_signal(sem, inc=1, device_id=(nbr,),
                                device_id_type=pl.DeviceIdType.MESH)
        pl.semaphore_wait(sem, 2)


def kernel(x, Wq, Wk, Wv, Wo):
    cosf, sin_a, sin_b = _rope_tables()
    bf = jnp.bfloat16
    args = (
        x.astype(bf).reshape(R, D),
        (Wq * SCALE).astype(bf),
        Wk.astype(bf),
        Wv.astype(bf),
        Wo.astype(bf),
        jnp.asarray(cosf, dtype=bf),
        jnp.asarray(sin_a, dtype=bf),
        jnp.asarray(sin_b, dtype=bf),
    )
    out = pl.pallas_call(
        _body,
        out_shape=jax.ShapeDtypeStruct((B, SQ, D), bf),
        in_specs=[pl.BlockSpec(memory_space=pltpu.VMEM)] * len(args),
        out_specs=pl.BlockSpec(memory_space=pltpu.VMEM),
        scratch_shapes=[
            pltpu.VMEM((N_DEV - 1, R, D), jnp.bfloat16),
            pltpu.VMEM((2 * N_DEV, SQ, D), jnp.bfloat16),
            pltpu.VMEM((6, SQ, D), jnp.bfloat16),
            pltpu.VMEM((R, D), jnp.bfloat16),
            pltpu.VMEM((R, D), jnp.bfloat16),
            pltpu.VMEM((R, D), jnp.bfloat16),
            pltpu.VMEM((R, D), jnp.bfloat16),
            pltpu.SemaphoreType.DMA((N_DEV - 1,)),
            pltpu.SemaphoreType.DMA((N_DEV - 1,)),
            pltpu.SemaphoreType.DMA((6,)),
            pltpu.SemaphoreType.DMA((6,)),
        ],
        compiler_params=pltpu.CompilerParams(
            collective_id=0,
            vmem_limit_bytes=100 * 1024 * 1024,
        ),
    )(*args)
    return out


if __name__ == "__main__":
    t = np.random.randn(R, D).astype(np.float64)
    cos, sa, sb = _rope_tables()
    tl = np.concatenate([t[:, 1:], t[:, :1]], axis=1)
    tr = np.concatenate([t[:, -1:], t[:, :-1]], axis=1)
    mine = t * cos + tl * sa + tr * sb

    t4 = t.reshape(B, SQ, HL, DH)
    inv = 1.0 / (10000.0 ** (np.arange(0, DH, 2) / DH))
    pos = np.arange(SQ)[:, None] * inv[None, :]
    cos_r = np.repeat(np.cos(pos), 2, axis=-1)
    sin_r = np.repeat(np.sin(pos), 2, axis=-1)
    t2 = t4.reshape(B, SQ, HL, DH // 2, 2)
    t_r = np.stack([-t2[..., 1], t2[..., 0]], axis=-1).reshape(B, SQ, HL, DH)
    ref = t4 * cos_r[None, :, None, :] + t_r * sin_r[None, :, None, :]
    print("rope table max err:", np.abs(mine - ref.reshape(R, D)).max())


# device time: 136922 ns/iter; 1.0347x vs baseline; 1.0095x over previous
import functools

import numpy as np
import jax
import jax.numpy as jnp
from jax import lax
from jax.experimental import pallas as pl
from jax.experimental.pallas import tpu as pltpu

N_DEV = 4
B, SQ, D = 2, 512, 1024
R = B * SQ
HL = 8
DH = 128
SCALE = 0.08838834764831843


def _rope_tables():
    inv = 1.0 / (10000.0 ** (np.arange(0, DH, 2) / DH))
    pos = np.arange(SQ)[:, None] * inv[None, :]
    cos = np.repeat(np.cos(pos), 2, axis=-1)
    sin = np.repeat(np.sin(pos), 2, axis=-1)
    even = (np.arange(DH) % 2 == 0).astype(np.float64)
    sin_a = -sin * even
    sin_b = sin * (1.0 - even)
    tile = lambda t: np.tile(t, (B, HL))
    return tile(cos), tile(sin_a), tile(sin_b)


def _body(x_ref, wq_ref, wk_ref, wv_ref, wo_ref, cos_ref, sa_ref, sb_ref,
          out_ref, xg, pout, rsbuf, qb, kb, vb, ctxb,
          ag_send, ag_recv, rs_send, rs_recv):
    my = lax.axis_index("i")
    right = lax.rem(my + 1, N_DEV)
    left = lax.rem(my + N_DEV - 1, N_DEV)

    bar = pltpu.get_barrier_semaphore()
    for nbr in (left, right):
        pl.semaphore_signal(bar, inc=1, device_id=(nbr,),
                            device_id_type=pl.DeviceIdType.MESH)
    pl.semaphore_wait(bar, 2)


    def ag_hop(k, src, dst, dev):
        return pltpu.make_async_remote_copy(
            src_ref=src,
            dst_ref=dst,
            send_sem=ag_send.at[k],
            recv_sem=ag_recv.at[k],
            device_id=(dev,),
            device_id_type=pl.DeviceIdType.MESH,
        )

    def rs_hop(s, j, c):
        k = 2 * s + j
        return pltpu.make_async_remote_copy(
            src_ref=pout.at[2 * c + j],
            dst_ref=rsbuf.at[k],
            send_sem=rs_send.at[k],
            recv_sem=rs_recv.at[k],
            device_id=(left,),
            device_id_type=pl.DeviceIdType.MESH,
        )

    def rope(t):
        tl = jnp.concatenate([t[:, 1:], t[:, :1]], axis=1)
        tr = jnp.concatenate([t[:, -1:], t[:, :-1]], axis=1)
        return t * cos_ref[...] + tl * sa_ref[...] + tr * sb_ref[...]

    def qkv_attn(xc):
        q = jnp.dot(xc, wq_ref[...],
                    preferred_element_type=jnp.float32).astype(jnp.bfloat16)
        qb[...] = rope(q)
        k = jnp.dot(xc, wk_ref[...],
                    preferred_element_type=jnp.float32).astype(jnp.bfloat16)
        kb[...] = rope(k)
        vb[...] = jnp.dot(
            xc, wv_ref[...],
            preferred_element_type=jnp.float32).astype(jnp.bfloat16)

        def attn_b(b, _):
            rs = pl.ds(b * SQ, SQ)
            for h in range(HL):
                cs = pl.ds(h * DH, DH)
                qh = qb[rs, cs]
                kh = kb[rs, cs]
                s = lax.dot_general(
                    qh, kh, (((1,), (1,)), ((), ())),
                    preferred_element_type=jnp.float32,
                )
                w = jnp.exp(s)
                r = 1.0 / jnp.sum(w, axis=1, keepdims=True)
                ctx = jnp.dot(w.astype(jnp.bfloat16), vb[rs, cs],
                              preferred_element_type=jnp.float32)
                ctxb[rs, cs] = (ctx * r).astype(jnp.bfloat16)
            return 0

        lax.fori_loop(0, B, attn_b, 0)

    def oproj_half(j):
        rows = pl.ds(j * SQ, SQ)
        return jnp.dot(ctxb[rows, :], wo_ref[...],
                       preferred_element_type=jnp.float32).astype(jnp.bfloat16)

    def rs_chunk(c, s, prev):
        descs = []
        for j in (0, 1):
            ph = oproj_half(j)
            if prev is not None:
                prev[j].wait_recv()
                ph = ph + rsbuf[2 * (s - 1) + j]
            pout[2 * c + j] = ph
            d = rs_hop(s, j, c)
            d.start()
            descs.append(d)
        return descs

    a0 = ag_hop(0, x_ref.at[pl.ds(0, SQ)], xg.at[0, pl.ds(0, SQ)], right)
    a1 = ag_hop(1, x_ref.at[pl.ds(SQ, SQ)], xg.at[0, pl.ds(SQ, SQ)], right)
    ag_b = ag_hop(2, x_ref, xg.at[1], left)
    a0.start()
    a1.start()
    ag_b.start()
    qkv_attn(x_ref[...])
    pout[6] = oproj_half(0)
    pout[7] = oproj_half(1)

    a0.wait_recv()
    c0 = ag_hop(3, xg.at[0, pl.ds(0, SQ)], xg.at[2, pl.ds(0, SQ)], right)
    c0.start()
    a1.wait_recv()
    c1 = ag_hop(4, xg.at[0, pl.ds(SQ, SQ)], xg.at[2, pl.ds(SQ, SQ)], right)
    c1.start()
    ag_b.wait_recv()
    qkv_attn(xg[1])
    rs0 = rs_chunk(1, 0, None)

    c0.wait_recv()
    c1.wait_recv()
    qkv_attn(xg[2])
    rs1 = rs_chunk(2, 1, rs0)

    qkv_attn(xg[0])
    rs2 = rs_chunk(0, 2, rs1)

    rs2[0].wait_recv()
    out_ref[0] = pout[6] + rsbuf[4]
    rs2[1].wait_recv()
    out_ref[1] = pout[7] + rsbuf[5]

    for d in (a0, a1, ag_b, c0, c1, *rs0, *rs1, *rs2):
        d.wait_send()

    @functools.partial(pl.run_scoped, sem=pltpu.SemaphoreType.REGULAR)
    def _(sem):
        for nbr in (left, right):
            pl.semaphore_signal(sem, inc=1, device_id=(nbr,),
                                device_id_type=pl.DeviceIdType.MESH)
        pl.semaphore_wait(sem, 2)


def kernel(x, Wq, Wk, Wv, Wo):
    cosf, sin_a, sin_b = _rope_tables()
    bf = jnp.bfloat16
    args = (
        x.astype(bf).reshape(R, D),
        (Wq * SCALE).astype(bf),
        Wk.astype(bf),
        Wv.astype(bf),
        Wo.astype(bf),
        jnp.asarray(cosf, dtype=bf),
        jnp.asarray(sin_a, dtype=bf),
        jnp.asarray(sin_b, dtype=bf),
    )
    out = pl.pallas_call(
        _body,
        out_shape=jax.ShapeDtypeStruct((B, SQ, D), bf),
        in_specs=[pl.BlockSpec(memory_space=pltpu.VMEM)] * len(args),
        out_specs=pl.BlockSpec(memory_space=pltpu.VMEM),
        scratch_shapes=[
            pltpu.VMEM((N_DEV - 1, R, D), jnp.bfloat16),
            pltpu.VMEM((2 * N_DEV, SQ, D), jnp.bfloat16),
            pltpu.VMEM((6, SQ, D), jnp.bfloat16),
            pltpu.VMEM((R, D), jnp.bfloat16),
            pltpu.VMEM((R, D), jnp.bfloat16),
            pltpu.VMEM((R, D), jnp.bfloat16),
            pltpu.VMEM((R, D), jnp.bfloat16),
            pltpu.SemaphoreType.DMA((5,)),
            pltpu.SemaphoreType.DMA((5,)),
            pltpu.SemaphoreType.DMA((6,)),
            pltpu.SemaphoreType.DMA((6,)),
        ],
        compiler_params=pltpu.CompilerParams(
            collective_id=0,
            vmem_limit_bytes=100 * 1024 * 1024,
        ),
    )(*args)
    return out


if __name__ == "__main__":
    t = np.random.randn(R, D).astype(np.float64)
    cos, sa, sb = _rope_tables()
    tl = np.concatenate([t[:, 1:], t[:, :1]], axis=1)
    tr = np.concatenate([t[:, -1:], t[:, :-1]], axis=1)
    mine = t * cos + tl * sa + tr * sb

    t4 = t.reshape(B, SQ, HL, DH)
    inv = 1.0 / (10000.0 ** (np.arange(0, DH, 2) / DH))
    pos = np.arange(SQ)[:, None] * inv[None, :]
    cos_r = np.repeat(np.cos(pos), 2, axis=-1)
    sin_r = np.repeat(np.sin(pos), 2, axis=-1)
    t2 = t4.reshape(B, SQ, HL, DH // 2, 2)
    t_r = np.stack([-t2[..., 1], t2[..., 0]], axis=-1).reshape(B, SQ, HL, DH)
    ref = t4 * cos_r[None, :, None, :] + t_r * sin_r[None, :, None, :]
    print("rope table max err:", np.abs(mine - ref.reshape(R, D)).max())


# device time: 136889 ns/iter; 1.0349x vs baseline; 1.0002x over previous
import functools

import numpy as np
import jax
import jax.numpy as jnp
from jax import lax
from jax.experimental import pallas as pl
from jax.experimental.pallas import tpu as pltpu

N_DEV = 4
B, SQ, D = 2, 512, 1024
R = B * SQ
HL = 8
DH = 128
SCALE = 0.08838834764831843


def _rope_tables():
    inv = 1.0 / (10000.0 ** (np.arange(0, DH, 2) / DH))
    pos = np.arange(SQ)[:, None] * inv[None, :]
    cos = np.repeat(np.cos(pos), 2, axis=-1)
    sin = np.repeat(np.sin(pos), 2, axis=-1)
    even = (np.arange(DH) % 2 == 0).astype(np.float64)
    sin_a = -sin * even
    sin_b = sin * (1.0 - even)
    tile = lambda t: np.tile(t, (B, HL))
    return tile(cos), tile(sin_a), tile(sin_b)


def _body(x_ref, wq_ref, wk_ref, wv_ref, wo_ref, cos_ref, sa_ref, sb_ref,
          out_ref, xg, pout, rsbuf, qb, kb, vb, ctxb,
          ag_send, ag_recv, rs_send, rs_recv):
    my = lax.axis_index("i")
    right = lax.rem(my + 1, N_DEV)
    left = lax.rem(my + N_DEV - 1, N_DEV)

    bar = pltpu.get_barrier_semaphore()
    for nbr in (left, right):
        pl.semaphore_signal(bar, inc=1, device_id=(nbr,),
                            device_id_type=pl.DeviceIdType.MESH)
    pl.semaphore_wait(bar, 2)


    def ag_hop(k, src, dst, dev):
        return pltpu.make_async_remote_copy(
            src_ref=src,
            dst_ref=dst,
            send_sem=ag_send.at[k],
            recv_sem=ag_recv.at[k],
            device_id=(dev,),
            device_id_type=pl.DeviceIdType.MESH,
        )

    def rs_hop(s, j, c):
        k = 2 * s + j
        return pltpu.make_async_remote_copy(
            src_ref=pout.at[2 * c + j],
            dst_ref=rsbuf.at[k],
            send_sem=rs_send.at[k],
            recv_sem=rs_recv.at[k],
            device_id=(left,),
            device_id_type=pl.DeviceIdType.MESH,
        )

    def rope(t):
        tl = jnp.concatenate([t[:, 1:], t[:, :1]], axis=1)
        tr = jnp.concatenate([t[:, -1:], t[:, :-1]], axis=1)
        return t * cos_ref[...] + tl * sa_ref[...] + tr * sb_ref[...]

    def qkv_attn(xc):
        q = jnp.dot(xc, wq_ref[...],
                    preferred_element_type=jnp.float32).astype(jnp.bfloat16)
        qb[...] = rope(q)
        k = jnp.dot(xc, wk_ref[...],
                    preferred_element_type=jnp.float32).astype(jnp.bfloat16)
        kb[...] = rope(k)
        vb[...] = jnp.dot(
            xc, wv_ref[...],
            preferred_element_type=jnp.float32).astype(jnp.bfloat16)

        def attn_b(b, _):
            rs = pl.ds(b * SQ, SQ)
            for h in range(HL):
                cs = pl.ds(h * DH, DH)
                qh = qb[rs, cs]
                kh = kb[rs, cs]
                s = lax.dot_general(
                    qh, kh, (((1,), (1,)), ((), ())),
                    preferred_element_type=jnp.float32,
                )
                w = jnp.exp(s)
                r = 1.0 / jnp.sum(w, axis=1, keepdims=True)
                ctx = jnp.dot(w.astype(jnp.bfloat16), vb[rs, cs],
                              preferred_element_type=jnp.float32)
                ctxb[rs, cs] = (ctx * r).astype(jnp.bfloat16)
            return 0

        lax.fori_loop(0, B, attn_b, 0)

    def oproj_half(j):
        rows = pl.ds(j * SQ, SQ)
        return jnp.dot(ctxb[rows, :], wo_ref[...],
                       preferred_element_type=jnp.float32).astype(jnp.bfloat16)

    def rs_chunk(c, s, prev):
        descs = []
        for j in (0, 1):
            ph = oproj_half(j)
            if prev is not None:
                prev[j].wait_recv()
                ph = ph + rsbuf[2 * (s - 1) + j]
            pout[2 * c + j] = ph
            d = rs_hop(s, j, c)
            d.start()
            descs.append(d)
        return descs

    a0 = ag_hop(0, x_ref.at[pl.ds(0, SQ)], xg.at[0, pl.ds(0, SQ)], right)
    a1 = ag_hop(1, x_ref.at[pl.ds(SQ, SQ)], xg.at[0, pl.ds(SQ, SQ)], right)
    ag_b = ag_hop(2, x_ref, xg.at[1], left)
    a0.start()
    ag_b.start()
    a1.start()
    qkv_attn(x_ref[...])
    pout[6] = oproj_half(0)
    pout[7] = oproj_half(1)

    a0.wait_recv()
    c0 = ag_hop(3, xg.at[0, pl.ds(0, SQ)], xg.at[2, pl.ds(0, SQ)], right)
    c0.start()
    a1.wait_recv()
    c1 = ag_hop(4, xg.at[0, pl.ds(SQ, SQ)], xg.at[2, pl.ds(SQ, SQ)], right)
    c1.start()
    ag_b.wait_recv()
    qkv_attn(xg[1])
    rs0 = rs_chunk(1, 0, None)

    c0.wait_recv()
    c1.wait_recv()
    qkv_attn(xg[2])
    rs1 = rs_chunk(2, 1, rs0)

    qkv_attn(xg[0])
    rs2 = rs_chunk(0, 2, rs1)

    rs2[0].wait_recv()
    out_ref[0] = pout[6] + rsbuf[4]
    rs2[1].wait_recv()
    out_ref[1] = pout[7] + rsbuf[5]

    for d in (a0, a1, ag_b, c0, c1, *rs0, *rs1, *rs2):
        d.wait_send()

    @functools.partial(pl.run_scoped, sem=pltpu.SemaphoreType.REGULAR)
    def _(sem):
        for nbr in (left, right):
            pl.semaphore_signal(sem, inc=1, device_id=(nbr,),
                                device_id_type=pl.DeviceIdType.MESH)
        pl.semaphore_wait(sem, 2)


def kernel(x, Wq, Wk, Wv, Wo):
    cosf, sin_a, sin_b = _rope_tables()
    bf = jnp.bfloat16
    args = (
        x.astype(bf).reshape(R, D),
        (Wq * SCALE).astype(bf),
        Wk.astype(bf),
        Wv.astype(bf),
        Wo.astype(bf),
        jnp.asarray(cosf, dtype=bf),
        jnp.asarray(sin_a, dtype=bf),
        jnp.asarray(sin_b, dtype=bf),
    )
    out = pl.pallas_call(
        _body,
        out_shape=jax.ShapeDtypeStruct((B, SQ, D), bf),
        in_specs=[pl.BlockSpec(memory_space=pltpu.VMEM)] * len(args),
        out_specs=pl.BlockSpec(memory_space=pltpu.VMEM),
        scratch_shapes=[
            pltpu.VMEM((N_DEV - 1, R, D), jnp.bfloat16),
            pltpu.VMEM((2 * N_DEV, SQ, D), jnp.bfloat16),
            pltpu.VMEM((6, SQ, D), jnp.bfloat16),
            pltpu.VMEM((R, D), jnp.bfloat16),
            pltpu.VMEM((R, D), jnp.bfloat16),
            pltpu.VMEM((R, D), jnp.bfloat16),
            pltpu.VMEM((R, D), jnp.bfloat16),
            pltpu.SemaphoreType.DMA((5,)),
            pltpu.SemaphoreType.DMA((5,)),
            pltpu.SemaphoreType.DMA((6,)),
            pltpu.SemaphoreType.DMA((6,)),
        ],
        compiler_params=pltpu.CompilerParams(
            collective_id=0,
            vmem_limit_bytes=100 * 1024 * 1024,
        ),
    )(*args)
    return out


if __name__ == "__main__":
    t = np.random.randn(R, D).astype(np.float64)
    cos, sa, sb = _rope_tables()
    tl = np.concatenate([t[:, 1:], t[:, :1]], axis=1)
    tr = np.concatenate([t[:, -1:], t[:, :-1]], axis=1)
    mine = t * cos + tl * sa + tr * sb

    t4 = t.reshape(B, SQ, HL, DH)
    inv = 1.0 / (10000.0 ** (np.arange(0, DH, 2) / DH))
    pos = np.arange(SQ)[:, None] * inv[None, :]
    cos_r = np.repeat(np.cos(pos), 2, axis=-1)
    sin_r = np.repeat(np.sin(pos), 2, axis=-1)
    t2 = t4.reshape(B, SQ, HL, DH // 2, 2)
    t_r = np.stack([-t2[..., 1], t2[..., 0]], axis=-1).reshape(B, SQ, HL, DH)
    ref = t4 * cos_r[None, :, None, :] + t_r * sin_r[None, :, None, :]
    print("rope table max err:", np.abs(mine - ref.reshape(R, D)).max())
